# Initial kernel scaffold; baseline (speedup 1.0000x reference)
import jax
import jax.numpy as jnp
from jax.experimental import pallas as pl


def kernel(x, edge_index, edge_attr, W_edge, W_enc, b_enc, W_dec, b_dec):
    raise NotImplementedError("write your pallas kernel here")


# speedup vs baseline: 14.7632x; 14.7632x over previous
import functools

import jax
import jax.numpy as jnp
from jax import lax
from jax.experimental import pallas as pl
from jax.experimental.pallas import tpu as pltpu
from jax.experimental.pallas import tpu_sc as plsc

NN = 10000
NP = 10240
NE = 320000
D = 128
EW = 128
NWIN = NE // EW


def _deg_body(src_hbm, dst_hbm, degp_hbm, idx_v, ones_v, stage_v,
              degs_sh, degd_sh):
    c = lax.axis_index("c")
    s = lax.axis_index("s")
    wid = c * 16 + s
    for j in range(8):
        ones_v[pl.ds(j * 16, 16)] = jnp.ones((16,), jnp.float32)
    zed = NP // 16
    def zrow(r, _):
        stage_v[pl.ds(r * 16, 16)] = jnp.zeros((16,), jnp.float32)
        return 0
    lax.fori_loop(0, zed // 16, zrow, 0)
    pltpu.sync_copy(stage_v, degs_sh.at[pl.ds(s * zed, zed)])
    pltpu.sync_copy(stage_v, degd_sh.at[pl.ds(s * zed, zed)])
    plsc.subcore_barrier()

    nbase, nrem = NWIN // 32, NWIN % 32
    nw = nbase + (wid < nrem).astype(jnp.int32)

    def body(i, _):
        base = (wid + i * 32) * EW
        pltpu.sync_copy(src_hbm.at[pl.ds(base, EW)], idx_v)
        pltpu.sync_copy(ones_v, degs_sh.at[idx_v], add=True)
        pltpu.sync_copy(dst_hbm.at[pl.ds(base, EW)], idx_v)
        pltpu.sync_copy(ones_v, degd_sh.at[idx_v], add=True)
        return 0
    lax.fori_loop(0, nw, body, 0)
    plsc.subcore_barrier()

    pltpu.sync_copy(degs_sh.at[pl.ds(s * zed, zed)], stage_v)
    pltpu.sync_copy(stage_v, degp_hbm.at[c, 0, pl.ds(s * zed, zed)])
    pltpu.sync_copy(degd_sh.at[pl.ds(s * zed, zed)], stage_v)
    pltpu.sync_copy(stage_v, degp_hbm.at[c, 1, pl.ds(s * zed, zed)])


def _deg_call(src, dst):
    mesh = plsc.VectorSubcoreMesh(core_axis_name="c", subcore_axis_name="s")
    zed = NP // 16
    f = functools.partial(
        pl.kernel,
        out_type=jax.ShapeDtypeStruct((2, 2, NP), jnp.float32),
        mesh=mesh,
        compiler_params=pltpu.CompilerParams(needs_layout_passes=False),
        scratch_types=[
            pltpu.VMEM((EW,), jnp.int32),
            pltpu.VMEM((EW,), jnp.float32),
            pltpu.VMEM((zed,), jnp.float32),
            pltpu.VMEM_SHARED((NP,), jnp.float32),
            pltpu.VMEM_SHARED((NP,), jnp.float32),
        ],
    )(_deg_body)
    return f(src, dst)


def _enc_body(x_ref, we_ref, ea_ref, wedge_ref, degp_ref,
              y_ref, ew_ref, inv_ref):
    y_ref[...] = jnp.dot(x_ref[...], we_ref[...],
                         preferred_element_type=jnp.float32)
    ea = ea_ref[...]
    z = (ea[0] * wedge_ref[0, 0] + ea[1] * wedge_ref[1, 0]
         + ea[2] * wedge_ref[2, 0] + ea[3] * wedge_ref[3, 0])
    ew_ref[...] = jax.nn.sigmoid(z)
    deg = degp_ref[...]
    degsum = deg[0] + deg[1]
    inv = jnp.where(degsum > 0,
                    lax.rsqrt(jnp.maximum(degsum, 1e-12)),
                    jnp.zeros_like(degsum))
    inv_ref[...] = inv


def _enc_call(x, W_enc, ea_T, W_edge, degp):
    return pl.pallas_call(
        _enc_body,
        out_shape=[
            jax.ShapeDtypeStruct((NN, D), jnp.float32),
            jax.ShapeDtypeStruct((NWIN, EW), jnp.float32),
            jax.ShapeDtypeStruct((2, NP), jnp.float32),
        ],
        in_specs=[
            pl.BlockSpec(memory_space=pltpu.VMEM),
            pl.BlockSpec(memory_space=pltpu.VMEM),
            pl.BlockSpec(memory_space=pltpu.VMEM),
            pl.BlockSpec(memory_space=pltpu.SMEM),
            pl.BlockSpec(memory_space=pltpu.VMEM),
        ],
    )(x, W_enc, ea_T, W_edge, degp)


def _agg_body(y_hbm, src_hbm, dst_hbm, ew_hbm, invs_hbm, invd_hbm, out_hbm,
              sidx, didx, ewv, invs_v, invd_v, rows, agg_sh, sem):
    c = lax.axis_index("c")
    s = lax.axis_index("s")
    wid = c * 16 + s

    pltpu.sync_copy(invs_hbm, invs_v)
    pltpu.sync_copy(invd_hbm, invd_v)

    def zrow(r, _):
        for j in range(8):
            rows[r, pl.ds(j * 16, 16)] = jnp.zeros((16,), jnp.float32)
        return 0
    lax.fori_loop(0, EW, zrow, 0)
    for t in range(NP // 16 // EW):
        pltpu.sync_copy(rows, agg_sh.at[pl.ds((s * 5 + t) * EW, EW)])
    plsc.subcore_barrier()

    nbase, nrem = NWIN // 32, NWIN % 32
    nw = nbase + (wid < nrem).astype(jnp.int32)

    def body(i, _):
        base = (wid + i * 32) * EW
        pltpu.sync_copy(src_hbm.at[pl.ds(base, EW)], sidx)
        pltpu.sync_copy(dst_hbm.at[pl.ds(base, EW)], didx)
        pltpu.sync_copy(ew_hbm.at[pl.ds(base, EW)], ewv)
        pltpu.async_copy(y_hbm.at[sidx], rows, sem).wait()

        def scale(cc, _):
            sl = pl.ds(cc * 16, 16)
            gs = plsc.load_gather(invs_v, [sidx[sl]])
            gd = plsc.load_gather(invd_v, [didx[sl]])
            co = gs * gd * ewv[sl]
            for l in range(16):
                r = cc * 16 + l
                cb = jnp.full((16,), co[l], jnp.float32)
                for j in range(8):
                    sl2 = pl.ds(j * 16, 16)
                    rows[r, sl2] = rows[r, sl2] * cb
            return 0
        lax.fori_loop(0, 8, scale, 0)
        pltpu.sync_copy(rows, agg_sh.at[didx], add=True)
        return 0
    lax.fori_loop(0, nw, body, 0)
    plsc.subcore_barrier()

    for t in range(NP // 16 // EW):
        off = (s * 5 + t) * EW
        pltpu.sync_copy(agg_sh.at[pl.ds(off, EW)], rows)
        pltpu.sync_copy(rows, out_hbm.at[c, pl.ds(off, EW)])


def _agg_call(y, src, dst, ewf, inv_s, inv_d):
    mesh = plsc.VectorSubcoreMesh(core_axis_name="c", subcore_axis_name="s")
    f = functools.partial(
        pl.kernel,
        out_type=jax.ShapeDtypeStruct((2, NP, D), jnp.float32),
        mesh=mesh,
        compiler_params=pltpu.CompilerParams(needs_layout_passes=False),
        scratch_types=[
            pltpu.VMEM((EW,), jnp.int32),
            pltpu.VMEM((EW,), jnp.int32),
            pltpu.VMEM((EW,), jnp.float32),
            pltpu.VMEM((NP,), jnp.float32),
            pltpu.VMEM((NP,), jnp.float32),
            pltpu.VMEM((EW, D), jnp.float32),
            pltpu.VMEM_SHARED((NP, D), jnp.float32),
            pltpu.SemaphoreType.DMA,
        ],
    )(_agg_body)
    return f(y, src, dst, ewf, inv_s, inv_d)


def _dec_body(aggp_ref, benc_ref, wd_ref, bdec_ref, out_ref):
    z = aggp_ref[0] + aggp_ref[1] + benc_ref[...]
    h = z * jax.nn.sigmoid(z)
    out_ref[...] = (jnp.dot(h, wd_ref[...], preferred_element_type=jnp.float32)
                    + bdec_ref[...])


def _dec_call(aggp, b_enc, W_dec, b_dec):
    return pl.pallas_call(
        _dec_body,
        out_shape=jax.ShapeDtypeStruct((NP, D), jnp.float32),
    )(aggp, b_enc, W_dec, b_dec)


def kernel(x, edge_index, edge_attr, W_edge, W_enc, b_enc, W_dec, b_dec):
    src = edge_index[0].astype(jnp.int32)
    dst = edge_index[1].astype(jnp.int32)
    ea_T = edge_attr.T.reshape(4, NWIN, EW)

    degp = _deg_call(src, dst)
    y, ew, inv2 = _enc_call(x, W_enc, ea_T, W_edge, degp)
    aggp = _agg_call(y, src, dst, ew.reshape(-1), inv2[0], inv2[1])
    out = _dec_call(aggp, b_enc, W_dec, b_dec)
    return out[:NN]
